# baseline (speedup 1.0000x reference)
import functools

import jax
import jax.numpy as jnp
import numpy as np
from jax import lax
from jax.experimental import pallas as pl
from jax.experimental.pallas import tpu as pltpu
from jax.experimental.pallas import tpu_sc as plsc

_R, _C = 4096, 1024
_N = _R * _C
_CHUNK = 256
_NCH = _R // _CHUNK
_INT_MIN = np.int32(-(2 ** 31))

_NB = 8192
_NCORE, _NSUB, _L = 2, 16, 16
_NW = _NCORE * _NSUB
_PER_W = _N // _NW
_PIECE = 16384
_NPIECE = _PER_W // _PIECE
_SLICE = _NB // _NSUB

_mesh = plsc.VectorSubcoreMesh(core_axis_name="c", subcore_axis_name="s")


def _biased_key(x):
    b = lax.bitcast_convert_type(x, jnp.int32)
    return jnp.bitwise_xor(
        b, jnp.bitwise_or(lax.shift_right_arithmetic(b, 31), _INT_MIN)
    )


def _hist_pass(x_hbm, t_hbm, out_hbm, xbuf, tbuf, hist, mbuf, shared,
               bucket_of, mask_extra):
    cid = lax.axis_index("c")
    sid = lax.axis_index("s")
    wid = sid * _NCORE + cid
    base = wid * _PER_W

    def zero_body(i, _):
        hist[pl.ds(i * _L, _L)] = jnp.zeros((_L,), jnp.int32)
        return 0

    lax.fori_loop(0, _NB // _L, zero_body, 0)

    ones = jnp.ones((_L,), jnp.int32)

    def piece_body(p, _):
        pltpu.sync_copy(x_hbm.at[pl.ds(base + p * _PIECE, _PIECE)], xbuf)
        pltpu.sync_copy(t_hbm.at[pl.ds(base + p * _PIECE, _PIECE)], tbuf)

        def vec_body(i, _):
            x = xbuf[pl.ds(i * _L, _L)]
            t = tbuf[pl.ds(i * _L, _L)]
            ub = _biased_key(x)
            neg = t == 0.0
            msk = jnp.logical_and(neg, mask_extra(ub))
            plsc.addupdate_scatter(hist, [bucket_of(ub)], ones, mask=msk)
            return 0

        lax.fori_loop(0, _PIECE // _L, vec_body, 0)
        return 0

    lax.fori_loop(0, _NPIECE, piece_body, 0)

    pltpu.sync_copy(hist, shared.at[sid])
    plsc.subcore_barrier()

    def fetch_row(r, _):
        pltpu.sync_copy(shared.at[r, pl.ds(sid * _SLICE, _SLICE)], mbuf.at[r])
        return 0

    lax.fori_loop(0, _NSUB, fetch_row, 0)

    def red_body(i, _):
        def row_body(r, a):
            return a + mbuf[r, pl.ds(i * _L, _L)]

        acc = lax.fori_loop(0, _NSUB, row_body, jnp.zeros((_L,), jnp.int32))
        hist[pl.ds(sid * _SLICE + i * _L, _L)] = acc
        return 0

    lax.fori_loop(0, _SLICE // _L, red_body, 0)
    pltpu.sync_copy(
        hist.at[pl.ds(sid * _SLICE, _SLICE)],
        out_hbm.at[cid, pl.ds(sid * _SLICE, _SLICE)],
    )


@functools.partial(
    pl.kernel,
    mesh=_mesh,
    out_type=jax.ShapeDtypeStruct((_NCORE, _NB), jnp.int32),
    compiler_params=pltpu.CompilerParams(needs_layout_passes=False),
    scratch_types=[
        pltpu.VMEM((_PIECE,), jnp.float32),
        pltpu.VMEM((_PIECE,), jnp.float32),
        pltpu.VMEM((_NB,), jnp.int32),
        pltpu.VMEM((_NSUB, _SLICE), jnp.int32),
        pltpu.VMEM_SHARED((_NSUB, _NB), jnp.int32),
    ],
)
def _sc_hist1(x_hbm, t_hbm, out_hbm, xbuf, tbuf, hist, mbuf, shared):
    _hist_pass(
        x_hbm, t_hbm, out_hbm, xbuf, tbuf, hist, mbuf, shared,
        bucket_of=lambda ub: lax.shift_right_logical(ub, 19),
        mask_extra=lambda ub: jnp.ones((_L,), jnp.bool_),
    )


@functools.partial(
    pl.kernel,
    mesh=_mesh,
    out_type=jax.ShapeDtypeStruct((_NCORE, _NB), jnp.int32),
    compiler_params=pltpu.CompilerParams(needs_layout_passes=False),
    scratch_types=[
        pltpu.VMEM((_PIECE,), jnp.float32),
        pltpu.VMEM((_PIECE,), jnp.float32),
        pltpu.VMEM((_NB,), jnp.int32),
        pltpu.VMEM((_NSUB, _SLICE), jnp.int32),
        pltpu.VMEM_SHARED((_NSUB, _NB), jnp.int32),
        pltpu.VMEM((_NB,), jnp.int32),
        pltpu.VMEM((_NB,), jnp.int32),
    ],
)
def _sc_hist2(x_hbm, t_hbm, h1_hbm, out_hbm, xbuf, tbuf, hist, mbuf, shared,
              h1a, h1b):
    pltpu.sync_copy(h1_hbm.at[0], h1a)
    pltpu.sync_copy(h1_hbm.at[1], h1b)

    def tot_body(i, acc):
        return acc + jnp.sum(h1a[pl.ds(i * _L, _L)] + h1b[pl.ds(i * _L, _L)])

    neg_cnt = lax.fori_loop(0, _NB // _L, tot_body, jnp.int32(0))
    pos_cnt = jnp.int32(_N) - neg_cnt
    k = (pos_cnt.astype(jnp.float32) * 3.0).astype(jnp.int32)
    k_eff = jnp.minimum(k, neg_cnt)

    def scan_body(ci, carry):
        cum, b1 = carry
        cc = _NB // _L - 1 - ci
        chunk = h1a[pl.ds(cc * _L, _L)] + h1b[pl.ds(cc * _L, _L)]
        ctotal = jnp.sum(chunk)
        suffix_in = lax.rev(jnp.cumsum(lax.rev(chunk, (0,))), (0,))
        c_t = jnp.sum((suffix_in + cum >= k_eff).astype(jnp.int32))
        crossed = jnp.logical_and(cum < k_eff, cum + ctotal >= k_eff)
        b1 = jnp.where(crossed, cc * _L + c_t - 1, b1)
        return cum + ctotal, b1

    _, b1 = lax.fori_loop(
        0, _NB // _L, scan_body, (jnp.int32(0), jnp.int32(_NB - 1))
    )
    b1v = jnp.full((_L,), b1, jnp.int32)

    _hist_pass(
        x_hbm, t_hbm, out_hbm, xbuf, tbuf, hist, mbuf, shared,
        bucket_of=lambda ub: jnp.bitwise_and(
            lax.shift_right_logical(ub, 6), np.int32(0x1FFF)
        ),
        mask_extra=lambda ub: lax.shift_right_logical(ub, 19) == b1v,
    )


def _find_bucket(m, k_val, iota):
    def bit_body(i, b):
        cand = jnp.bitwise_or(b, jnp.left_shift(jnp.int32(1), 12 - i))
        cnt = jnp.sum(jnp.where(iota >= cand, m, 0))
        return jnp.where(cnt >= k_val, cand, b)

    return lax.fori_loop(0, 13, bit_body, jnp.int32(0))


def _tc_final_body(x_ref, t_ref, h1_ref, h2_ref, o_ref):
    iota = lax.broadcasted_iota(jnp.int32, (1, _NB), 1)
    m1 = h1_ref[0:1, :] + h1_ref[1:2, :]
    m2 = h2_ref[0:1, :] + h2_ref[1:2, :]
    neg_cnt = jnp.sum(m1)
    pos_cnt = jnp.int32(_N) - neg_cnt
    k = (pos_cnt.astype(jnp.float32) * 3.0).astype(jnp.int32)
    k_eff = jnp.minimum(k, neg_cnt)

    b1 = _find_bucket(m1, k_eff, iota)
    cnt_gt1 = jnp.sum(jnp.where(iota > b1, m1, 0))
    k_rem = k_eff - cnt_gt1
    b2 = _find_bucket(m2, k_rem, iota)

    thr = jnp.bitwise_xor(
        jnp.bitwise_or(jnp.left_shift(b1, 19), jnp.left_shift(b2, 6)),
        _INT_MIN,
    )

    def final_chunk(i, carry):
        pos_loss, cnt_gt, sum_gt, cnt_eq, sum_eq = carry
        x = x_ref[pl.ds(i * _CHUNK, _CHUNK), :]
        t = t_ref[pl.ds(i * _CHUNK, _CHUNK), :]
        pos = t > 0.0
        neg = jnp.logical_not(pos)
        sp = jnp.maximum(x, 0.0) + jnp.log1p(jnp.exp(-jnp.abs(x)))
        pos_loss = pos_loss + jnp.sum(jnp.where(pos, sp - x * t, 0.0))
        b = lax.bitcast_convert_type(x, jnp.int32)
        s = jnp.where(b >= 0, b, jnp.bitwise_xor(jnp.bitwise_not(b), _INT_MIN))
        s_pref = jnp.bitwise_and(s, np.int32(-64))
        gt = jnp.logical_and(neg, s_pref > thr)
        eq = jnp.logical_and(neg, s_pref == thr)
        cnt_gt = cnt_gt + jnp.sum(gt.astype(jnp.int32))
        sum_gt = sum_gt + jnp.sum(jnp.where(gt, sp, 0.0))
        cnt_eq = cnt_eq + jnp.sum(eq.astype(jnp.int32))
        sum_eq = sum_eq + jnp.sum(jnp.where(eq, sp, 0.0))
        return pos_loss, cnt_gt, sum_gt, cnt_eq, sum_eq

    pos_loss, cnt_gt, sum_gt, cnt_eq, sum_eq = lax.fori_loop(
        0, _NCH, final_chunk,
        (jnp.float32(0.0), jnp.int32(0), jnp.float32(0.0), jnp.int32(0),
         jnp.float32(0.0)),
    )

    tie_cnt = k_eff - cnt_gt
    sp_thr = jnp.where(cnt_eq > 0, sum_eq / cnt_eq.astype(jnp.float32), 0.0)
    tie = jnp.where(tie_cnt > 0, tie_cnt.astype(jnp.float32) * sp_thr, 0.0)
    total = pos_loss + sum_gt + tie
    denom = (pos_cnt + k).astype(jnp.float32)
    o_ref[0, 0] = total / denom


def kernel(input, target):
    xf = input.reshape(_N)
    tf = target.reshape(_N)
    h1 = _sc_hist1(xf, tf)
    h2 = _sc_hist2(xf, tf, h1)
    x2 = input.reshape(_R, _C)
    t2 = target.reshape(_R, _C)
    out = pl.pallas_call(
        _tc_final_body,
        out_shape=jax.ShapeDtypeStruct((1, 1), jnp.float32),
        in_specs=[
            pl.BlockSpec((_R, _C), lambda: (0, 0)),
            pl.BlockSpec((_R, _C), lambda: (0, 0)),
            pl.BlockSpec((_NCORE, _NB), lambda: (0, 0)),
            pl.BlockSpec((_NCORE, _NB), lambda: (0, 0)),
        ],
        out_specs=pl.BlockSpec(memory_space=pltpu.SMEM),
    )(x2, t2, h1, h2)
    return out[0, 0]


# speedup vs baseline: 1.1429x; 1.1429x over previous
import functools

import jax
import jax.numpy as jnp
import numpy as np
from jax import lax
from jax.experimental import pallas as pl
from jax.experimental.pallas import tpu as pltpu
from jax.experimental.pallas import tpu_sc as plsc

_ROWS, _COLS = 128, 32768
_N = _ROWS * _COLS
_CCH = 2048
_NCCH = _COLS // _CCH
_INT_MIN = np.int32(-(2 ** 31))

_NB = 8192
_NCORE, _NSUB, _L = 2, 16, 16
_NW = _NCORE * _NSUB
_PER_W = _N // _NW
_PIECE = 32768
_NPIECE = _PER_W // _PIECE
_UNROLL = 4
_SLICE = _NB // _NSUB

_mesh = plsc.VectorSubcoreMesh(core_axis_name="c", subcore_axis_name="s")


def _hist_pass(x_hbm, t_hbm, out_hbm, xbuf, tbuf, hist, mbuf, shared,
               bucket_of, mask_of):
    cid = lax.axis_index("c")
    sid = lax.axis_index("s")
    wid = sid * _NCORE + cid
    base = wid * _PER_W

    def zero_body(i, _):
        hist[pl.ds(i * _L, _L)] = jnp.zeros((_L,), jnp.int32)
        return 0

    lax.fori_loop(0, _NB // _L, zero_body, 0)

    ones = jnp.ones((_L,), jnp.int32)

    def piece_body(p, _):
        pltpu.sync_copy(x_hbm.at[pl.ds(base + p * _PIECE, _PIECE)], xbuf)
        pltpu.sync_copy(t_hbm.at[pl.ds(base + p * _PIECE, _PIECE)], tbuf)

        def vec_body(i, _):
            for u in range(_UNROLL):
                off = (i * _UNROLL + u) * _L
                x = xbuf[pl.ds(off, _L)]
                t = tbuf[pl.ds(off, _L)]
                b = lax.bitcast_convert_type(x, jnp.int32)
                neg = t == 0.0
                plsc.addupdate_scatter(
                    hist, [bucket_of(b)], ones, mask=mask_of(b, neg)
                )
            return 0

        lax.fori_loop(0, _PIECE // (_L * _UNROLL), vec_body, 0)
        return 0

    lax.fori_loop(0, _NPIECE, piece_body, 0)

    pltpu.sync_copy(hist, shared.at[sid])
    plsc.subcore_barrier()

    def fetch_row(r, _):
        pltpu.sync_copy(shared.at[r, pl.ds(sid * _SLICE, _SLICE)], mbuf.at[r])
        return 0

    lax.fori_loop(0, _NSUB, fetch_row, 0)

    def red_body(i, _):
        def row_body(r, a):
            return a + mbuf[r, pl.ds(i * _L, _L)]

        acc = lax.fori_loop(0, _NSUB, row_body, jnp.zeros((_L,), jnp.int32))
        hist[pl.ds(sid * _SLICE + i * _L, _L)] = acc
        return 0

    lax.fori_loop(0, _SLICE // _L, red_body, 0)
    pltpu.sync_copy(
        hist.at[pl.ds(sid * _SLICE, _SLICE)],
        out_hbm.at[cid, pl.ds(sid * _SLICE, _SLICE)],
    )


@functools.partial(
    pl.kernel,
    mesh=_mesh,
    out_type=jax.ShapeDtypeStruct((_NCORE, _NB), jnp.int32),
    compiler_params=pltpu.CompilerParams(needs_layout_passes=False),
    scratch_types=[
        pltpu.VMEM((_PIECE,), jnp.float32),
        pltpu.VMEM((_PIECE,), jnp.float32),
        pltpu.VMEM((_NB,), jnp.int32),
        pltpu.VMEM((_NSUB, _SLICE), jnp.int32),
        pltpu.VMEM_SHARED((_NSUB, _NB), jnp.int32),
    ],
)
def _sc_hist1(x_hbm, t_hbm, out_hbm, xbuf, tbuf, hist, mbuf, shared):
    _hist_pass(
        x_hbm, t_hbm, out_hbm, xbuf, tbuf, hist, mbuf, shared,
        bucket_of=lambda b: lax.shift_right_logical(b, 19),
        mask_of=lambda b, neg: neg,
    )


@functools.partial(
    pl.kernel,
    mesh=_mesh,
    out_type=jax.ShapeDtypeStruct((_NCORE, _NB), jnp.int32),
    compiler_params=pltpu.CompilerParams(needs_layout_passes=False),
    scratch_types=[
        pltpu.VMEM((_PIECE,), jnp.float32),
        pltpu.VMEM((_PIECE,), jnp.float32),
        pltpu.VMEM((_NB,), jnp.int32),
        pltpu.VMEM((_NSUB, _SLICE), jnp.int32),
        pltpu.VMEM_SHARED((_NSUB, _NB), jnp.int32),
        pltpu.VMEM((_NB,), jnp.int32),
        pltpu.VMEM((_NB,), jnp.int32),
    ],
)
def _sc_hist2(x_hbm, t_hbm, h1_hbm, out_hbm, xbuf, tbuf, hist, mbuf, shared,
              h1a, h1b):
    pltpu.sync_copy(h1_hbm.at[0], h1a)
    pltpu.sync_copy(h1_hbm.at[1], h1b)

    def tot_body(i, acc):
        return acc + jnp.sum(h1a[pl.ds(i * _L, _L)] + h1b[pl.ds(i * _L, _L)])

    neg_cnt = lax.fori_loop(0, _NB // _L, tot_body, jnp.int32(0))
    pos_cnt = jnp.int32(_N) - neg_cnt
    k = (pos_cnt.astype(jnp.float32) * 3.0).astype(jnp.int32)
    k_eff = jnp.minimum(k, neg_cnt)

    _HALF = _NB // (2 * _L)

    def scan_body(si, carry):
        cum, b1 = carry
        is_pos = si < _HALF
        cc = jnp.where(is_pos, _HALF - 1 - si, si)
        chunk = h1a[pl.ds(cc * _L, _L)] + h1b[pl.ds(cc * _L, _L)]
        ctotal = jnp.sum(chunk)
        suffix_in = lax.rev(jnp.cumsum(lax.rev(chunk, (0,))), (0,))
        pre_in = jnp.cumsum(chunk)
        vals = jnp.where(is_pos, suffix_in, pre_in)
        c_t = jnp.sum((vals + cum >= k_eff).astype(jnp.int32))
        cand = jnp.where(is_pos, cc * _L + c_t - 1, cc * _L + _L - c_t)
        crossed = jnp.logical_and(cum < k_eff, cum + ctotal >= k_eff)
        b1 = jnp.where(crossed, cand, b1)
        return cum + ctotal, b1

    _, b1 = lax.fori_loop(
        0, 2 * _HALF, scan_body, (jnp.int32(0), jnp.int32(_NB // 2 - 1))
    )
    b1v = jnp.full((_L,), b1, jnp.int32)

    _hist_pass(
        x_hbm, t_hbm, out_hbm, xbuf, tbuf, hist, mbuf, shared,
        bucket_of=lambda b: jnp.bitwise_and(
            lax.shift_right_logical(b, 6), np.int32(0x1FFF)
        ),
        mask_of=lambda b, neg: jnp.logical_and(
            neg, lax.shift_right_logical(b, 19) == b1v
        ),
    )


def _find_bucket(m, k_val, rank):
    def bit_body(i, b):
        cand = jnp.bitwise_or(b, jnp.left_shift(jnp.int32(1), 12 - i))
        cnt = jnp.sum(jnp.where(rank >= cand, m, 0))
        return jnp.where(cnt >= k_val, cand, b)

    return lax.fori_loop(0, 13, bit_body, jnp.int32(0))


def _tc_pos_body(x_ref, t_ref, o_ref):
    def chunk(i, acc):
        x = x_ref[:, pl.ds(i * _CCH, _CCH)]
        t = t_ref[:, pl.ds(i * _CCH, _CCH)]
        sp = jnp.maximum(x, 0.0) + jnp.log1p(jnp.exp(-jnp.abs(x)))
        return acc + jnp.sum(jnp.where(t > 0.0, sp - x * t, 0.0))

    o_ref[0, 0] = lax.fori_loop(0, _NCCH, chunk, jnp.float32(0.0))


def _tc_final_body(x_ref, t_ref, h1_ref, h2_ref, pos_ref, o_ref):
    iota = lax.broadcasted_iota(jnp.int32, (1, _NB), 1)
    m1 = h1_ref[0:1, :] + h1_ref[1:2, :]
    m2 = h2_ref[0:1, :] + h2_ref[1:2, :]
    neg_cnt = jnp.sum(m1)
    pos_cnt = jnp.int32(_N) - neg_cnt
    k = (pos_cnt.astype(jnp.float32) * 3.0).astype(jnp.int32)
    k_eff = jnp.minimum(k, neg_cnt)

    a1 = jnp.bitwise_xor(
        iota, jnp.where(iota < _NB // 2, np.int32(0x1000), np.int32(0x1FFF))
    )
    b1a = _find_bucket(m1, k_eff, a1)
    cnt_gt1 = jnp.sum(jnp.where(a1 > b1a, m1, 0))
    k_rem = k_eff - cnt_gt1
    a2 = jnp.where(b1a >= _NB // 2, iota, np.int32(_NB - 1) - iota)
    b2a = _find_bucket(m2, k_rem, a2)

    thr = jnp.bitwise_xor(
        jnp.bitwise_or(jnp.left_shift(b1a, 19), jnp.left_shift(b2a, 6)),
        _INT_MIN,
    )

    def final_chunk(i, carry):
        cnt_gt, sum_gt, cnt_eq, sum_eq = carry
        x = x_ref[:, pl.ds(i * _CCH, _CCH)]
        t = t_ref[:, pl.ds(i * _CCH, _CCH)]
        neg = t == 0.0
        sp = jnp.maximum(x, 0.0) + jnp.log1p(jnp.exp(-jnp.abs(x)))
        b = lax.bitcast_convert_type(x, jnp.int32)
        s = jnp.where(b >= 0, b, jnp.bitwise_xor(jnp.bitwise_not(b), _INT_MIN))
        s_pref = jnp.bitwise_and(s, np.int32(-64))
        gt = jnp.logical_and(neg, s_pref > thr)
        eq = jnp.logical_and(neg, s_pref == thr)
        cnt_gt = cnt_gt + jnp.sum(gt.astype(jnp.int32))
        sum_gt = sum_gt + jnp.sum(jnp.where(gt, sp, 0.0))
        cnt_eq = cnt_eq + jnp.sum(eq.astype(jnp.int32))
        sum_eq = sum_eq + jnp.sum(jnp.where(eq, sp, 0.0))
        return cnt_gt, sum_gt, cnt_eq, sum_eq

    cnt_gt, sum_gt, cnt_eq, sum_eq = lax.fori_loop(
        0, _NCCH, final_chunk,
        (jnp.int32(0), jnp.float32(0.0), jnp.int32(0), jnp.float32(0.0)),
    )

    tie_cnt = k_eff - cnt_gt
    sp_thr = jnp.where(cnt_eq > 0, sum_eq / cnt_eq.astype(jnp.float32), 0.0)
    tie = jnp.where(tie_cnt > 0, tie_cnt.astype(jnp.float32) * sp_thr, 0.0)
    total = pos_ref[0, 0] + sum_gt + tie
    denom = (pos_cnt + k).astype(jnp.float32)
    o_ref[0, 0] = total / denom


def kernel(input, target):
    xf = input.reshape(_N)
    tf = target.reshape(_N)
    h1 = _sc_hist1(xf, tf)
    h2 = _sc_hist2(xf, tf, h1)
    pos_part = pl.pallas_call(
        _tc_pos_body,
        out_shape=jax.ShapeDtypeStruct((1, 1), jnp.float32),
        in_specs=[
            pl.BlockSpec((_ROWS, _COLS), lambda: (0, 0)),
            pl.BlockSpec((_ROWS, _COLS), lambda: (0, 0)),
        ],
        out_specs=pl.BlockSpec(memory_space=pltpu.SMEM),
    )(input, target)
    out = pl.pallas_call(
        _tc_final_body,
        out_shape=jax.ShapeDtypeStruct((1, 1), jnp.float32),
        in_specs=[
            pl.BlockSpec((_ROWS, _COLS), lambda: (0, 0)),
            pl.BlockSpec((_ROWS, _COLS), lambda: (0, 0)),
            pl.BlockSpec((_NCORE, _NB), lambda: (0, 0)),
            pl.BlockSpec((_NCORE, _NB), lambda: (0, 0)),
            pl.BlockSpec(memory_space=pltpu.SMEM),
        ],
        out_specs=pl.BlockSpec(memory_space=pltpu.SMEM),
    )(input, target, h1, h2, pos_part)
    return out[0, 0]


# speedup vs baseline: 1.2857x; 1.1250x over previous
import functools

import jax
import jax.numpy as jnp
import numpy as np
from jax import lax
from jax.experimental import pallas as pl
from jax.experimental.pallas import tpu as pltpu
from jax.experimental.pallas import tpu_sc as plsc

_ROWS, _COLS = 128, 32768
_N = _ROWS * _COLS
_CCH = 2048
_NCCH = _COLS // _CCH
_INT_MIN = np.int32(-(2 ** 31))

_NB = 8192
_NCORE, _NSUB, _L = 2, 16, 16
_NW = _NCORE * _NSUB
_ROWS_PER_W = _ROWS // _NW
_PIECE = _COLS
_UNROLL = 4
_SLICE = _NB // _NSUB

_mesh = plsc.VectorSubcoreMesh(core_axis_name="c", subcore_axis_name="s")


def _hist_pass(x_hbm, t_hbm, out_hbm, xbuf, tbuf, hist, mbuf, shared,
               bucket_of, mask_of):
    cid = lax.axis_index("c")
    sid = lax.axis_index("s")
    wid = sid * _NCORE + cid
    base_row = wid * _ROWS_PER_W

    def zero_body(i, _):
        hist[pl.ds(i * _L, _L)] = jnp.zeros((_L,), jnp.int32)
        return 0

    lax.fori_loop(0, _NB // _L, zero_body, 0)

    ones = jnp.ones((_L,), jnp.int32)

    def piece_body(p, _):
        pltpu.sync_copy(x_hbm.at[base_row + p], xbuf)
        pltpu.sync_copy(t_hbm.at[base_row + p], tbuf)

        def vec_body(i, _):
            for u in range(_UNROLL):
                off = (i * _UNROLL + u) * _L
                x = xbuf[pl.ds(off, _L)]
                t = tbuf[pl.ds(off, _L)]
                b = lax.bitcast_convert_type(x, jnp.int32)
                neg = t == 0.0
                plsc.addupdate_scatter(
                    hist, [bucket_of(b)], ones, mask=mask_of(b, neg)
                )
            return 0

        lax.fori_loop(0, _PIECE // (_L * _UNROLL), vec_body, 0)
        return 0

    lax.fori_loop(0, _ROWS_PER_W, piece_body, 0)

    pltpu.sync_copy(hist, shared.at[sid])
    plsc.subcore_barrier()

    def fetch_row(r, _):
        pltpu.sync_copy(shared.at[r, pl.ds(sid * _SLICE, _SLICE)], mbuf.at[r])
        return 0

    lax.fori_loop(0, _NSUB, fetch_row, 0)

    def red_body(i, _):
        def row_body(r, a):
            return a + mbuf[r, pl.ds(i * _L, _L)]

        acc = lax.fori_loop(0, _NSUB, row_body, jnp.zeros((_L,), jnp.int32))
        hist[pl.ds(sid * _SLICE + i * _L, _L)] = acc
        return 0

    lax.fori_loop(0, _SLICE // _L, red_body, 0)
    pltpu.sync_copy(
        hist.at[pl.ds(sid * _SLICE, _SLICE)],
        out_hbm.at[cid, pl.ds(sid * _SLICE, _SLICE)],
    )


@functools.partial(
    pl.kernel,
    mesh=_mesh,
    out_type=jax.ShapeDtypeStruct((_NCORE, _NB), jnp.int32),
    compiler_params=pltpu.CompilerParams(needs_layout_passes=False),
    scratch_types=[
        pltpu.VMEM((_PIECE,), jnp.float32),
        pltpu.VMEM((_PIECE,), jnp.float32),
        pltpu.VMEM((_NB,), jnp.int32),
        pltpu.VMEM((_NSUB, _SLICE), jnp.int32),
        pltpu.VMEM_SHARED((_NSUB, _NB), jnp.int32),
    ],
)
def _sc_hist1(x_hbm, t_hbm, out_hbm, xbuf, tbuf, hist, mbuf, shared):
    _hist_pass(
        x_hbm, t_hbm, out_hbm, xbuf, tbuf, hist, mbuf, shared,
        bucket_of=lambda b: lax.shift_right_logical(b, 19),
        mask_of=lambda b, neg: neg,
    )


@functools.partial(
    pl.kernel,
    mesh=_mesh,
    out_type=jax.ShapeDtypeStruct((_NCORE, _NB), jnp.int32),
    compiler_params=pltpu.CompilerParams(needs_layout_passes=False),
    scratch_types=[
        pltpu.VMEM((_PIECE,), jnp.float32),
        pltpu.VMEM((_PIECE,), jnp.float32),
        pltpu.VMEM((_NB,), jnp.int32),
        pltpu.VMEM((_NSUB, _SLICE), jnp.int32),
        pltpu.VMEM_SHARED((_NSUB, _NB), jnp.int32),
        pltpu.VMEM((_NB,), jnp.int32),
        pltpu.VMEM((_NB,), jnp.int32),
    ],
)
def _sc_hist2(x_hbm, t_hbm, h1_hbm, out_hbm, xbuf, tbuf, hist, mbuf, shared,
              h1a, h1b):
    pltpu.sync_copy(h1_hbm.at[0], h1a)
    pltpu.sync_copy(h1_hbm.at[1], h1b)

    def tot_body(i, acc):
        return acc + jnp.sum(h1a[pl.ds(i * _L, _L)] + h1b[pl.ds(i * _L, _L)])

    neg_cnt = lax.fori_loop(0, _NB // _L, tot_body, jnp.int32(0))
    pos_cnt = jnp.int32(_N) - neg_cnt
    k = (pos_cnt.astype(jnp.float32) * 3.0).astype(jnp.int32)
    k_eff = jnp.minimum(k, neg_cnt)

    _HALF = _NB // (2 * _L)

    def scan_body(si, carry):
        cum, b1 = carry
        is_pos = si < _HALF
        cc = jnp.where(is_pos, _HALF - 1 - si, si)
        chunk = h1a[pl.ds(cc * _L, _L)] + h1b[pl.ds(cc * _L, _L)]
        ctotal = jnp.sum(chunk)
        suffix_in = lax.rev(jnp.cumsum(lax.rev(chunk, (0,))), (0,))
        pre_in = jnp.cumsum(chunk)
        vals = jnp.where(is_pos, suffix_in, pre_in)
        c_t = jnp.sum((vals + cum >= k_eff).astype(jnp.int32))
        cand = jnp.where(is_pos, cc * _L + c_t - 1, cc * _L + _L - c_t)
        crossed = jnp.logical_and(cum < k_eff, cum + ctotal >= k_eff)
        b1 = jnp.where(crossed, cand, b1)
        return cum + ctotal, b1

    _, b1 = lax.fori_loop(
        0, 2 * _HALF, scan_body, (jnp.int32(0), jnp.int32(_NB // 2 - 1))
    )
    b1v = jnp.full((_L,), b1, jnp.int32)

    _hist_pass(
        x_hbm, t_hbm, out_hbm, xbuf, tbuf, hist, mbuf, shared,
        bucket_of=lambda b: jnp.bitwise_and(
            lax.shift_right_logical(b, 6), np.int32(0x1FFF)
        ),
        mask_of=lambda b, neg: jnp.logical_and(
            neg, lax.shift_right_logical(b, 19) == b1v
        ),
    )


def _find_bucket(m, k_val, rank):
    def bit_body(i, b):
        cand = jnp.bitwise_or(b, jnp.left_shift(jnp.int32(1), 12 - i))
        cnt = jnp.sum(jnp.where(rank >= cand, m, 0))
        return jnp.where(cnt >= k_val, cand, b)

    return lax.fori_loop(0, 13, bit_body, jnp.int32(0))


def _tc_pos_body(x_ref, t_ref, o_ref):
    def chunk(i, acc):
        x = x_ref[:, pl.ds(i * _CCH, _CCH)]
        t = t_ref[:, pl.ds(i * _CCH, _CCH)]
        sp = jnp.maximum(x, 0.0) + jnp.log1p(jnp.exp(-jnp.abs(x)))
        return acc + jnp.sum(jnp.where(t > 0.0, sp - x * t, 0.0))

    o_ref[0, 0] = lax.fori_loop(0, _NCCH, chunk, jnp.float32(0.0))


def _tc_final_body(x_ref, t_ref, h1_ref, h2_ref, pos_ref, o_ref):
    iota = lax.broadcasted_iota(jnp.int32, (1, _NB), 1)
    m1 = h1_ref[0:1, :] + h1_ref[1:2, :]
    m2 = h2_ref[0:1, :] + h2_ref[1:2, :]
    neg_cnt = jnp.sum(m1)
    pos_cnt = jnp.int32(_N) - neg_cnt
    k = (pos_cnt.astype(jnp.float32) * 3.0).astype(jnp.int32)
    k_eff = jnp.minimum(k, neg_cnt)

    a1 = jnp.bitwise_xor(
        iota, jnp.where(iota < _NB // 2, np.int32(0x1000), np.int32(0x1FFF))
    )
    b1a = _find_bucket(m1, k_eff, a1)
    cnt_gt1 = jnp.sum(jnp.where(a1 > b1a, m1, 0))
    k_rem = k_eff - cnt_gt1
    a2 = jnp.where(b1a >= _NB // 2, iota, np.int32(_NB - 1) - iota)
    b2a = _find_bucket(m2, k_rem, a2)

    thr = jnp.bitwise_xor(
        jnp.bitwise_or(jnp.left_shift(b1a, 19), jnp.left_shift(b2a, 6)),
        _INT_MIN,
    )

    def final_chunk(i, carry):
        cnt_gt, sum_gt, cnt_eq, sum_eq = carry
        x = x_ref[:, pl.ds(i * _CCH, _CCH)]
        t = t_ref[:, pl.ds(i * _CCH, _CCH)]
        neg = t == 0.0
        sp = jnp.maximum(x, 0.0) + jnp.log1p(jnp.exp(-jnp.abs(x)))
        b = lax.bitcast_convert_type(x, jnp.int32)
        s = jnp.where(b >= 0, b, jnp.bitwise_xor(jnp.bitwise_not(b), _INT_MIN))
        s_pref = jnp.bitwise_and(s, np.int32(-64))
        gt = jnp.logical_and(neg, s_pref > thr)
        eq = jnp.logical_and(neg, s_pref == thr)
        cnt_gt = cnt_gt + jnp.sum(gt.astype(jnp.int32))
        sum_gt = sum_gt + jnp.sum(jnp.where(gt, sp, 0.0))
        cnt_eq = cnt_eq + jnp.sum(eq.astype(jnp.int32))
        sum_eq = sum_eq + jnp.sum(jnp.where(eq, sp, 0.0))
        return cnt_gt, sum_gt, cnt_eq, sum_eq

    cnt_gt, sum_gt, cnt_eq, sum_eq = lax.fori_loop(
        0, _NCCH, final_chunk,
        (jnp.int32(0), jnp.float32(0.0), jnp.int32(0), jnp.float32(0.0)),
    )

    tie_cnt = k_eff - cnt_gt
    sp_thr = jnp.where(cnt_eq > 0, sum_eq / cnt_eq.astype(jnp.float32), 0.0)
    tie = jnp.where(tie_cnt > 0, tie_cnt.astype(jnp.float32) * sp_thr, 0.0)
    total = pos_ref[0, 0] + sum_gt + tie
    denom = (pos_cnt + k).astype(jnp.float32)
    o_ref[0, 0] = total / denom


def kernel(input, target):
    h1 = _sc_hist1(input, target)
    h2 = _sc_hist2(input, target, h1)
    pos_part = pl.pallas_call(
        _tc_pos_body,
        out_shape=jax.ShapeDtypeStruct((1, 1), jnp.float32),
        in_specs=[
            pl.BlockSpec((_ROWS, _COLS), lambda: (0, 0)),
            pl.BlockSpec((_ROWS, _COLS), lambda: (0, 0)),
        ],
        out_specs=pl.BlockSpec(memory_space=pltpu.SMEM),
    )(input, target)
    out = pl.pallas_call(
        _tc_final_body,
        out_shape=jax.ShapeDtypeStruct((1, 1), jnp.float32),
        in_specs=[
            pl.BlockSpec((_ROWS, _COLS), lambda: (0, 0)),
            pl.BlockSpec((_ROWS, _COLS), lambda: (0, 0)),
            pl.BlockSpec((_NCORE, _NB), lambda: (0, 0)),
            pl.BlockSpec((_NCORE, _NB), lambda: (0, 0)),
            pl.BlockSpec(memory_space=pltpu.SMEM),
        ],
        out_specs=pl.BlockSpec(memory_space=pltpu.SMEM),
    )(input, target, h1, h2, pos_part)
    return out[0, 0]


# speedup vs baseline: 1.2991x; 1.0105x over previous
import functools

import jax
import jax.numpy as jnp
import numpy as np
from jax import lax
from jax.experimental import pallas as pl
from jax.experimental.pallas import tpu as pltpu
from jax.experimental.pallas import tpu_sc as plsc

_ROWS, _COLS = 128, 32768
_N = _ROWS * _COLS
_CCH = 2048
_NCCH = _COLS // _CCH
_INT_MIN = np.int32(-(2 ** 31))

_NB = 8192
_NCORE, _NSUB, _L = 2, 16, 16
_NW = _NCORE * _NSUB
_ROWS_PER_W = _ROWS // _NW
_PIECE = _COLS
_UNROLL = 8
_SLICE = _NB // _NSUB

_mesh = plsc.VectorSubcoreMesh(core_axis_name="c", subcore_axis_name="s")


def _hist_pass(x_hbm, t_hbm, out_hbm, xbuf, tbuf, hist, mbuf, shared,
               bucket_of, mask_of):
    cid = lax.axis_index("c")
    sid = lax.axis_index("s")
    wid = sid * _NCORE + cid
    base_row = wid * _ROWS_PER_W

    def zero_body(i, _):
        hist[pl.ds(i * _L, _L)] = jnp.zeros((_L,), jnp.int32)
        return 0

    lax.fori_loop(0, _NB // _L, zero_body, 0)

    ones = jnp.ones((_L,), jnp.int32)

    def piece_body(p, _):
        pltpu.sync_copy(x_hbm.at[base_row + p], xbuf)
        pltpu.sync_copy(t_hbm.at[base_row + p], tbuf)

        def vec_body(i, _):
            for u in range(_UNROLL):
                off = (i * _UNROLL + u) * _L
                x = xbuf[pl.ds(off, _L)]
                t = tbuf[pl.ds(off, _L)]
                b = lax.bitcast_convert_type(x, jnp.int32)
                neg = t == 0.0
                plsc.addupdate_scatter(
                    hist, [bucket_of(b)], ones, mask=mask_of(b, neg)
                )
            return 0

        lax.fori_loop(0, _PIECE // (_L * _UNROLL), vec_body, 0)
        return 0

    lax.fori_loop(0, _ROWS_PER_W, piece_body, 0)

    pltpu.sync_copy(hist, shared.at[sid])
    plsc.subcore_barrier()

    def fetch_row(r, _):
        pltpu.sync_copy(shared.at[r, pl.ds(sid * _SLICE, _SLICE)], mbuf.at[r])
        return 0

    lax.fori_loop(0, _NSUB, fetch_row, 0)

    def red_body(i, _):
        def row_body(r, a):
            return a + mbuf[r, pl.ds(i * _L, _L)]

        acc = lax.fori_loop(0, _NSUB, row_body, jnp.zeros((_L,), jnp.int32))
        hist[pl.ds(sid * _SLICE + i * _L, _L)] = acc
        return 0

    lax.fori_loop(0, _SLICE // _L, red_body, 0)
    pltpu.sync_copy(
        hist.at[pl.ds(sid * _SLICE, _SLICE)],
        out_hbm.at[cid, pl.ds(sid * _SLICE, _SLICE)],
    )


@functools.partial(
    pl.kernel,
    mesh=_mesh,
    out_type=jax.ShapeDtypeStruct((_NCORE, _NB), jnp.int32),
    compiler_params=pltpu.CompilerParams(needs_layout_passes=False),
    scratch_types=[
        pltpu.VMEM((_PIECE,), jnp.float32),
        pltpu.VMEM((_PIECE,), jnp.float32),
        pltpu.VMEM((_NB,), jnp.int32),
        pltpu.VMEM((_NSUB, _SLICE), jnp.int32),
        pltpu.VMEM_SHARED((_NSUB, _NB), jnp.int32),
    ],
)
def _sc_hist1(x_hbm, t_hbm, out_hbm, xbuf, tbuf, hist, mbuf, shared):
    _hist_pass(
        x_hbm, t_hbm, out_hbm, xbuf, tbuf, hist, mbuf, shared,
        bucket_of=lambda b: lax.shift_right_logical(b, 19),
        mask_of=lambda b, neg: neg,
    )


@functools.partial(
    pl.kernel,
    mesh=_mesh,
    out_type=jax.ShapeDtypeStruct((_NCORE, _NB), jnp.int32),
    compiler_params=pltpu.CompilerParams(needs_layout_passes=False),
    scratch_types=[
        pltpu.VMEM((_PIECE,), jnp.float32),
        pltpu.VMEM((_PIECE,), jnp.float32),
        pltpu.VMEM((_NB,), jnp.int32),
        pltpu.VMEM((_NSUB, _SLICE), jnp.int32),
        pltpu.VMEM_SHARED((_NSUB, _NB), jnp.int32),
        pltpu.VMEM((_NB,), jnp.int32),
        pltpu.VMEM((_NB,), jnp.int32),
    ],
)
def _sc_hist2(x_hbm, t_hbm, h1_hbm, out_hbm, xbuf, tbuf, hist, mbuf, shared,
              h1a, h1b):
    pltpu.sync_copy(h1_hbm.at[0], h1a)
    pltpu.sync_copy(h1_hbm.at[1], h1b)

    def tot_body(i, acc):
        return acc + jnp.sum(h1a[pl.ds(i * _L, _L)] + h1b[pl.ds(i * _L, _L)])

    neg_cnt = lax.fori_loop(0, _NB // _L, tot_body, jnp.int32(0))
    pos_cnt = jnp.int32(_N) - neg_cnt
    k = (pos_cnt.astype(jnp.float32) * 3.0).astype(jnp.int32)
    k_eff = jnp.minimum(k, neg_cnt)

    _HALF = _NB // (2 * _L)

    def scan_body(si, carry):
        cum, b1 = carry
        is_pos = si < _HALF
        cc = jnp.where(is_pos, _HALF - 1 - si, si)
        chunk = h1a[pl.ds(cc * _L, _L)] + h1b[pl.ds(cc * _L, _L)]
        ctotal = jnp.sum(chunk)
        suffix_in = lax.rev(jnp.cumsum(lax.rev(chunk, (0,))), (0,))
        pre_in = jnp.cumsum(chunk)
        vals = jnp.where(is_pos, suffix_in, pre_in)
        c_t = jnp.sum((vals + cum >= k_eff).astype(jnp.int32))
        cand = jnp.where(is_pos, cc * _L + c_t - 1, cc * _L + _L - c_t)
        crossed = jnp.logical_and(cum < k_eff, cum + ctotal >= k_eff)
        b1 = jnp.where(crossed, cand, b1)
        return cum + ctotal, b1

    _, b1 = lax.fori_loop(
        0, 2 * _HALF, scan_body, (jnp.int32(0), jnp.int32(_NB // 2 - 1))
    )
    b1v = jnp.full((_L,), b1, jnp.int32)

    _hist_pass(
        x_hbm, t_hbm, out_hbm, xbuf, tbuf, hist, mbuf, shared,
        bucket_of=lambda b: jnp.bitwise_and(
            lax.shift_right_logical(b, 6), np.int32(0x1FFF)
        ),
        mask_of=lambda b, neg: jnp.logical_and(
            neg, lax.shift_right_logical(b, 19) == b1v
        ),
    )


def _find_bucket(m, k_val, rank):
    def bit_body(i, b):
        cand = jnp.bitwise_or(b, jnp.left_shift(jnp.int32(1), 12 - i))
        cnt = jnp.sum(jnp.where(rank >= cand, m, 0))
        return jnp.where(cnt >= k_val, cand, b)

    return lax.fori_loop(0, 13, bit_body, jnp.int32(0))


def _tc_pos_body(x_ref, t_ref, o_ref):
    def chunk(i, acc):
        x = x_ref[:, pl.ds(i * _CCH, _CCH)]
        t = t_ref[:, pl.ds(i * _CCH, _CCH)]
        sp = jnp.maximum(x, 0.0) + jnp.log1p(jnp.exp(-jnp.abs(x)))
        return acc + jnp.sum(jnp.where(t > 0.0, sp - x * t, 0.0))

    o_ref[0, 0] = lax.fori_loop(0, _NCCH, chunk, jnp.float32(0.0))


def _tc_final_body(x_ref, t_ref, h1_ref, h2_ref, pos_ref, o_ref):
    iota = lax.broadcasted_iota(jnp.int32, (1, _NB), 1)
    m1 = h1_ref[0:1, :] + h1_ref[1:2, :]
    m2 = h2_ref[0:1, :] + h2_ref[1:2, :]
    neg_cnt = jnp.sum(m1)
    pos_cnt = jnp.int32(_N) - neg_cnt
    k = (pos_cnt.astype(jnp.float32) * 3.0).astype(jnp.int32)
    k_eff = jnp.minimum(k, neg_cnt)

    a1 = jnp.bitwise_xor(
        iota, jnp.where(iota < _NB // 2, np.int32(0x1000), np.int32(0x1FFF))
    )
    b1a = _find_bucket(m1, k_eff, a1)
    cnt_gt1 = jnp.sum(jnp.where(a1 > b1a, m1, 0))
    k_rem = k_eff - cnt_gt1
    a2 = jnp.where(b1a >= _NB // 2, iota, np.int32(_NB - 1) - iota)
    b2a = _find_bucket(m2, k_rem, a2)

    thr = jnp.bitwise_xor(
        jnp.bitwise_or(jnp.left_shift(b1a, 19), jnp.left_shift(b2a, 6)),
        _INT_MIN,
    )

    def final_chunk(i, carry):
        cnt_gt, sum_gt, cnt_eq, sum_eq = carry
        x = x_ref[:, pl.ds(i * _CCH, _CCH)]
        t = t_ref[:, pl.ds(i * _CCH, _CCH)]
        neg = t == 0.0
        sp = jnp.maximum(x, 0.0) + jnp.log1p(jnp.exp(-jnp.abs(x)))
        b = lax.bitcast_convert_type(x, jnp.int32)
        s = jnp.where(b >= 0, b, jnp.bitwise_xor(jnp.bitwise_not(b), _INT_MIN))
        s_pref = jnp.bitwise_and(s, np.int32(-64))
        gt = jnp.logical_and(neg, s_pref > thr)
        eq = jnp.logical_and(neg, s_pref == thr)
        cnt_gt = cnt_gt + jnp.sum(gt.astype(jnp.int32))
        sum_gt = sum_gt + jnp.sum(jnp.where(gt, sp, 0.0))
        cnt_eq = cnt_eq + jnp.sum(eq.astype(jnp.int32))
        sum_eq = sum_eq + jnp.sum(jnp.where(eq, sp, 0.0))
        return cnt_gt, sum_gt, cnt_eq, sum_eq

    cnt_gt, sum_gt, cnt_eq, sum_eq = lax.fori_loop(
        0, _NCCH, final_chunk,
        (jnp.int32(0), jnp.float32(0.0), jnp.int32(0), jnp.float32(0.0)),
    )

    tie_cnt = k_eff - cnt_gt
    sp_thr = jnp.where(cnt_eq > 0, sum_eq / cnt_eq.astype(jnp.float32), 0.0)
    tie = jnp.where(tie_cnt > 0, tie_cnt.astype(jnp.float32) * sp_thr, 0.0)
    total = pos_ref[0, 0] + sum_gt + tie
    denom = (pos_cnt + k).astype(jnp.float32)
    o_ref[0, 0] = total / denom


def kernel(input, target):
    h1 = _sc_hist1(input, target)
    h2 = _sc_hist2(input, target, h1)
    pos_part = pl.pallas_call(
        _tc_pos_body,
        out_shape=jax.ShapeDtypeStruct((1, 1), jnp.float32),
        in_specs=[
            pl.BlockSpec((_ROWS, _COLS), lambda: (0, 0)),
            pl.BlockSpec((_ROWS, _COLS), lambda: (0, 0)),
        ],
        out_specs=pl.BlockSpec(memory_space=pltpu.SMEM),
    )(input, target)
    out = pl.pallas_call(
        _tc_final_body,
        out_shape=jax.ShapeDtypeStruct((1, 1), jnp.float32),
        in_specs=[
            pl.BlockSpec((_ROWS, _COLS), lambda: (0, 0)),
            pl.BlockSpec((_ROWS, _COLS), lambda: (0, 0)),
            pl.BlockSpec((_NCORE, _NB), lambda: (0, 0)),
            pl.BlockSpec((_NCORE, _NB), lambda: (0, 0)),
            pl.BlockSpec(memory_space=pltpu.SMEM),
        ],
        out_specs=pl.BlockSpec(memory_space=pltpu.SMEM),
    )(input, target, h1, h2, pos_part)
    return out[0, 0]


# speedup vs baseline: 1.4243x; 1.0964x over previous
import functools

import jax
import jax.numpy as jnp
import numpy as np
from jax import lax
from jax.experimental import pallas as pl
from jax.experimental.pallas import tpu as pltpu
from jax.experimental.pallas import tpu_sc as plsc

_ROWS, _COLS = 128, 32768
_N = _ROWS * _COLS
_CCH = 2048
_NCCH = _COLS // _CCH
_INT_MIN = np.int32(-(2 ** 31))

_NB = 8192
_NCORE, _NSUB, _L = 2, 16, 16
_NW = _NCORE * _NSUB
_ROWS_PER_W = _ROWS // _NW
_PIECE = _COLS // 2
_NPIECE = _ROWS_PER_W * 2
_UNROLL = 8
_SLICE = _NB // _NSUB

_mesh = plsc.VectorSubcoreMesh(core_axis_name="c", subcore_axis_name="s")


def _hist_pass(x_hbm, t_hbm, out_hbm, xb0, xb1, tb0, tb1, sx0, sx1, st0, st1,
               hist, mbuf, shared, bucket_of, mask_of):
    cid = lax.axis_index("c")
    sid = lax.axis_index("s")
    wid = sid * _NCORE + cid
    base_row = wid * _ROWS_PER_W

    def zero_body(i, _):
        hist[pl.ds(i * _L, _L)] = jnp.zeros((_L,), jnp.int32)
        return 0

    lax.fori_loop(0, _NB // _L, zero_body, 0)

    ones = jnp.ones((_L,), jnp.int32)
    xb, tb = (xb0, xb1), (tb0, tb1)
    sx, st = (sx0, sx1), (st0, st1)

    def start(p):
        r = base_row + p // 2
        c = (p % 2) * _PIECE
        hx = pltpu.async_copy(x_hbm.at[r, pl.ds(c, _PIECE)], xb[p % 2],
                              sx[p % 2])
        ht = pltpu.async_copy(t_hbm.at[r, pl.ds(c, _PIECE)], tb[p % 2],
                              st[p % 2])
        return hx, ht

    def process(xbuf, tbuf):
        def vec_body(i, _):
            for u in range(_UNROLL):
                off = (i * _UNROLL + u) * _L
                x = xbuf[pl.ds(off, _L)]
                t = tbuf[pl.ds(off, _L)]
                b = lax.bitcast_convert_type(x, jnp.int32)
                neg = t == 0.0
                plsc.addupdate_scatter(
                    hist, [bucket_of(b)], ones, mask=mask_of(b, neg)
                )
            return 0

        lax.fori_loop(0, _PIECE // (_L * _UNROLL), vec_body, 0)

    pend = start(0)
    for p in range(_NPIECE):
        nxt = start(p + 1) if p + 1 < _NPIECE else None
        pend[0].wait()
        pend[1].wait()
        process(xb[p % 2], tb[p % 2])
        pend = nxt

    pltpu.sync_copy(hist, shared.at[sid])
    plsc.subcore_barrier()

    def fetch_row(r, _):
        pltpu.sync_copy(shared.at[r, pl.ds(sid * _SLICE, _SLICE)], mbuf.at[r])
        return 0

    lax.fori_loop(0, _NSUB, fetch_row, 0)

    def red_body(i, _):
        def row_body(r, a):
            return a + mbuf[r, pl.ds(i * _L, _L)]

        acc = lax.fori_loop(0, _NSUB, row_body, jnp.zeros((_L,), jnp.int32))
        hist[pl.ds(sid * _SLICE + i * _L, _L)] = acc
        return 0

    lax.fori_loop(0, _SLICE // _L, red_body, 0)
    pltpu.sync_copy(
        hist.at[pl.ds(sid * _SLICE, _SLICE)],
        out_hbm.at[cid, pl.ds(sid * _SLICE, _SLICE)],
    )


@functools.partial(
    pl.kernel,
    mesh=_mesh,
    out_type=jax.ShapeDtypeStruct((_NCORE, _NB), jnp.int32),
    compiler_params=pltpu.CompilerParams(needs_layout_passes=False),
    scratch_types=[
        pltpu.VMEM((_PIECE,), jnp.float32),
        pltpu.VMEM((_PIECE,), jnp.float32),
        pltpu.VMEM((_PIECE,), jnp.float32),
        pltpu.VMEM((_PIECE,), jnp.float32),
        pltpu.SemaphoreType.DMA,
        pltpu.SemaphoreType.DMA,
        pltpu.SemaphoreType.DMA,
        pltpu.SemaphoreType.DMA,
        pltpu.VMEM((_NB,), jnp.int32),
        pltpu.VMEM((_NSUB, _SLICE), jnp.int32),
        pltpu.VMEM_SHARED((_NSUB, _NB), jnp.int32),
    ],
)
def _sc_hist1(x_hbm, t_hbm, out_hbm, xb0, xb1, tb0, tb1, sx0, sx1, st0, st1,
              hist, mbuf, shared):
    _hist_pass(
        x_hbm, t_hbm, out_hbm, xb0, xb1, tb0, tb1, sx0, sx1, st0, st1,
        hist, mbuf, shared,
        bucket_of=lambda b: lax.shift_right_logical(b, 19),
        mask_of=lambda b, neg: neg,
    )


@functools.partial(
    pl.kernel,
    mesh=_mesh,
    out_type=jax.ShapeDtypeStruct((_NCORE, _NB), jnp.int32),
    compiler_params=pltpu.CompilerParams(needs_layout_passes=False),
    scratch_types=[
        pltpu.VMEM((_PIECE,), jnp.float32),
        pltpu.VMEM((_PIECE,), jnp.float32),
        pltpu.VMEM((_PIECE,), jnp.float32),
        pltpu.VMEM((_PIECE,), jnp.float32),
        pltpu.SemaphoreType.DMA,
        pltpu.SemaphoreType.DMA,
        pltpu.SemaphoreType.DMA,
        pltpu.SemaphoreType.DMA,
        pltpu.VMEM((_NB,), jnp.int32),
        pltpu.VMEM((_NSUB, _SLICE), jnp.int32),
        pltpu.VMEM_SHARED((_NSUB, _NB), jnp.int32),
        pltpu.VMEM((_NB,), jnp.int32),
        pltpu.VMEM((_NB,), jnp.int32),
    ],
)
def _sc_hist2(x_hbm, t_hbm, h1_hbm, out_hbm, xb0, xb1, tb0, tb1,
              sx0, sx1, st0, st1, hist, mbuf, shared, h1a, h1b):
    pltpu.sync_copy(h1_hbm.at[0], h1a)
    pltpu.sync_copy(h1_hbm.at[1], h1b)

    def tot_body(i, acc):
        return acc + jnp.sum(h1a[pl.ds(i * _L, _L)] + h1b[pl.ds(i * _L, _L)])

    neg_cnt = lax.fori_loop(0, _NB // _L, tot_body, jnp.int32(0))
    pos_cnt = jnp.int32(_N) - neg_cnt
    k = (pos_cnt.astype(jnp.float32) * 3.0).astype(jnp.int32)
    k_eff = jnp.minimum(k, neg_cnt)

    _HALF = _NB // (2 * _L)

    def scan_body(si, carry):
        cum, b1 = carry
        is_pos = si < _HALF
        cc = jnp.where(is_pos, _HALF - 1 - si, si)
        chunk = h1a[pl.ds(cc * _L, _L)] + h1b[pl.ds(cc * _L, _L)]
        ctotal = jnp.sum(chunk)
        suffix_in = lax.rev(jnp.cumsum(lax.rev(chunk, (0,))), (0,))
        pre_in = jnp.cumsum(chunk)
        vals = jnp.where(is_pos, suffix_in, pre_in)
        c_t = jnp.sum((vals + cum >= k_eff).astype(jnp.int32))
        cand = jnp.where(is_pos, cc * _L + c_t - 1, cc * _L + _L - c_t)
        crossed = jnp.logical_and(cum < k_eff, cum + ctotal >= k_eff)
        b1 = jnp.where(crossed, cand, b1)
        return cum + ctotal, b1

    _, b1 = lax.fori_loop(
        0, 2 * _HALF, scan_body, (jnp.int32(0), jnp.int32(_NB // 2 - 1))
    )
    b1v = jnp.full((_L,), b1, jnp.int32)

    _hist_pass(
        x_hbm, t_hbm, out_hbm, xb0, xb1, tb0, tb1, sx0, sx1, st0, st1,
        hist, mbuf, shared,
        bucket_of=lambda b: jnp.bitwise_and(
            lax.shift_right_logical(b, 6), np.int32(0x1FFF)
        ),
        mask_of=lambda b, neg: jnp.logical_and(
            neg, lax.shift_right_logical(b, 19) == b1v
        ),
    )


def _find_bucket(m, k_val, rank):
    def bit_body(i, b):
        cand = jnp.bitwise_or(b, jnp.left_shift(jnp.int32(1), 12 - i))
        cnt = jnp.sum(jnp.where(rank >= cand, m, 0))
        return jnp.where(cnt >= k_val, cand, b)

    return lax.fori_loop(0, 13, bit_body, jnp.int32(0))


def _tc_pos_body(x_ref, t_ref, o_ref):
    def chunk(i, acc):
        x = x_ref[:, pl.ds(i * _CCH, _CCH)]
        t = t_ref[:, pl.ds(i * _CCH, _CCH)]
        sp = jnp.maximum(x, 0.0) + jnp.log1p(jnp.exp(-jnp.abs(x)))
        return acc + jnp.sum(jnp.where(t > 0.0, sp - x * t, 0.0))

    o_ref[0, 0] = lax.fori_loop(0, _NCCH, chunk, jnp.float32(0.0))


def _tc_final_body(x_ref, t_ref, h1_ref, h2_ref, pos_ref, o_ref):
    iota = lax.broadcasted_iota(jnp.int32, (1, _NB), 1)
    m1 = h1_ref[0:1, :] + h1_ref[1:2, :]
    m2 = h2_ref[0:1, :] + h2_ref[1:2, :]
    neg_cnt = jnp.sum(m1)
    pos_cnt = jnp.int32(_N) - neg_cnt
    k = (pos_cnt.astype(jnp.float32) * 3.0).astype(jnp.int32)
    k_eff = jnp.minimum(k, neg_cnt)

    a1 = jnp.bitwise_xor(
        iota, jnp.where(iota < _NB // 2, np.int32(0x1000), np.int32(0x1FFF))
    )
    b1a = _find_bucket(m1, k_eff, a1)
    cnt_gt1 = jnp.sum(jnp.where(a1 > b1a, m1, 0))
    k_rem = k_eff - cnt_gt1
    a2 = jnp.where(b1a >= _NB // 2, iota, np.int32(_NB - 1) - iota)
    b2a = _find_bucket(m2, k_rem, a2)

    thr = jnp.bitwise_xor(
        jnp.bitwise_or(jnp.left_shift(b1a, 19), jnp.left_shift(b2a, 6)),
        _INT_MIN,
    )

    def final_chunk(i, carry):
        cnt_gt, sum_gt, cnt_eq, sum_eq = carry
        x = x_ref[:, pl.ds(i * _CCH, _CCH)]
        t = t_ref[:, pl.ds(i * _CCH, _CCH)]
        neg = t == 0.0
        sp = jnp.maximum(x, 0.0) + jnp.log1p(jnp.exp(-jnp.abs(x)))
        b = lax.bitcast_convert_type(x, jnp.int32)
        s = jnp.where(b >= 0, b, jnp.bitwise_xor(jnp.bitwise_not(b), _INT_MIN))
        s_pref = jnp.bitwise_and(s, np.int32(-64))
        gt = jnp.logical_and(neg, s_pref > thr)
        eq = jnp.logical_and(neg, s_pref == thr)
        cnt_gt = cnt_gt + jnp.sum(gt.astype(jnp.int32))
        sum_gt = sum_gt + jnp.sum(jnp.where(gt, sp, 0.0))
        cnt_eq = cnt_eq + jnp.sum(eq.astype(jnp.int32))
        sum_eq = sum_eq + jnp.sum(jnp.where(eq, sp, 0.0))
        return cnt_gt, sum_gt, cnt_eq, sum_eq

    cnt_gt, sum_gt, cnt_eq, sum_eq = lax.fori_loop(
        0, _NCCH, final_chunk,
        (jnp.int32(0), jnp.float32(0.0), jnp.int32(0), jnp.float32(0.0)),
    )

    tie_cnt = k_eff - cnt_gt
    sp_thr = jnp.where(cnt_eq > 0, sum_eq / cnt_eq.astype(jnp.float32), 0.0)
    tie = jnp.where(tie_cnt > 0, tie_cnt.astype(jnp.float32) * sp_thr, 0.0)
    total = pos_ref[0, 0] + sum_gt + tie
    denom = (pos_cnt + k).astype(jnp.float32)
    o_ref[0, 0] = total / denom


def kernel(input, target):
    h1 = _sc_hist1(input, target)
    h2 = _sc_hist2(input, target, h1)
    pos_part = pl.pallas_call(
        _tc_pos_body,
        out_shape=jax.ShapeDtypeStruct((1, 1), jnp.float32),
        in_specs=[
            pl.BlockSpec((_ROWS, _COLS), lambda: (0, 0)),
            pl.BlockSpec((_ROWS, _COLS), lambda: (0, 0)),
        ],
        out_specs=pl.BlockSpec(memory_space=pltpu.SMEM),
    )(input, target)
    out = pl.pallas_call(
        _tc_final_body,
        out_shape=jax.ShapeDtypeStruct((1, 1), jnp.float32),
        in_specs=[
            pl.BlockSpec((_ROWS, _COLS), lambda: (0, 0)),
            pl.BlockSpec((_ROWS, _COLS), lambda: (0, 0)),
            pl.BlockSpec((_NCORE, _NB), lambda: (0, 0)),
            pl.BlockSpec((_NCORE, _NB), lambda: (0, 0)),
            pl.BlockSpec(memory_space=pltpu.SMEM),
        ],
        out_specs=pl.BlockSpec(memory_space=pltpu.SMEM),
    )(input, target, h1, h2, pos_part)
    return out[0, 0]
